# baseline (device time: 151990 ns/iter reference)
import jax
import jax.numpy as jnp
from jax import lax
from jax.experimental import pallas as pl
from jax.experimental.pallas import tpu as pltpu

N_DEV = 4


def kernel(A, B):
    m, k = A.shape
    _, n = B.shape

    def body(a_ref, b_ref, out_ref, comm_ref, send_sems, recv_sems):
        my = lax.axis_index("i")
        left = (my - 1) % N_DEV
        right = (my + 1) % N_DEV

        barrier_sem = pltpu.get_barrier_semaphore()
        for nbr in [left, right]:
            pl.semaphore_signal(
                barrier_sem, inc=1,
                device_id=(nbr,), device_id_type=pl.DeviceIdType.MESH,
            )
        pl.semaphore_wait(barrier_sem, 2)

        partial = jnp.dot(a_ref[:, :], b_ref[:, :],
                          preferred_element_type=jnp.float32)
        comm_ref[0, :, :] = partial
        out_ref[:, :] = partial

        for h in range(N_DEV - 1):
            rdma = pltpu.make_async_remote_copy(
                src_ref=comm_ref.at[h],
                dst_ref=comm_ref.at[h + 1],
                send_sem=send_sems.at[h],
                recv_sem=recv_sems.at[h],
                device_id=(right,),
                device_id_type=pl.DeviceIdType.MESH,
            )
            rdma.start()
            rdma.wait()
            out_ref[:, :] = out_ref[:, :] + comm_ref[h + 1, :, :]

        z = out_ref[:, :]
        out_ref[:, :] = z / (1.0 + jnp.exp(-z))

    return pl.pallas_call(
        body,
        out_shape=jax.ShapeDtypeStruct((m, n), jnp.float32),
        in_specs=[
            pl.BlockSpec(memory_space=pltpu.VMEM),
            pl.BlockSpec(memory_space=pltpu.VMEM),
        ],
        out_specs=pl.BlockSpec(memory_space=pltpu.VMEM),
        scratch_shapes=[
            pltpu.VMEM((N_DEV, m, n), jnp.float32),
            pltpu.SemaphoreType.DMA((N_DEV - 1,)),
            pltpu.SemaphoreType.DMA((N_DEV - 1,)),
        ],
        compiler_params=pltpu.CompilerParams(collective_id=0),
    )(A, B)


# device time: 49199 ns/iter; 3.0893x vs baseline; 3.0893x over previous
import jax
import jax.numpy as jnp
from jax import lax
from jax.experimental import pallas as pl
from jax.experimental.pallas import tpu as pltpu

N_DEV = 4
C = 128


def kernel(A, B):
    m, k = A.shape
    _, n = B.shape

    def body(a_ref, b_ref, out_ref, z_ref, rs1_ref, rs2_ref, ss, rs_):
        me = lax.axis_index("i")
        p1 = me ^ 1
        p2 = 3 - me

        def qa(q):
            return q * 256

        def qb(q):
            return q * 256 + C

        def col(ref, start, w=C):
            return ref.at[:, pl.ds(start, w)]

        def mm_block(c):
            z_ref[:, pl.ds(c, C)] = jnp.dot(
                a_ref[:, :], b_ref[:, pl.ds(c, C)],
                preferred_element_type=jnp.float32)

        def add_block(c, src):
            z_ref[:, pl.ds(c, C)] = z_ref[:, pl.ds(c, C)] + src

        def send_to(i, src_sl, dst_sl, dev):
            d = pltpu.make_async_remote_copy(
                src_ref=src_sl, dst_ref=dst_sl,
                send_sem=ss.at[i], recv_sem=rs_.at[i],
                device_id=(dev,), device_id_type=pl.DeviceIdType.MESH,
            )
            d.start()
            return d

        def recv_at(i, dst_sl):
            return pltpu.make_async_remote_copy(
                src_ref=dst_sl, dst_ref=dst_sl,
                send_sem=ss.at[i], recv_sem=rs_.at[i],
                device_id=(me,), device_id_type=pl.DeviceIdType.MESH,
            )

        barrier_sem = pltpu.get_barrier_semaphore()
        for nbr in [p1, p2]:
            pl.semaphore_signal(
                barrier_sem, inc=1,
                device_id=(nbr,), device_id_type=pl.DeviceIdType.MESH,
            )
        pl.semaphore_wait(barrier_sem, 2)

        mm_block(qa(3 - p1))
        s1 = send_to(1, col(z_ref, qa(3 - p1)), rs1_ref.at[1], p1)
        mm_block(qb(p2 ^ 1))
        s3 = send_to(3, col(z_ref, qb(p2 ^ 1)), rs1_ref.at[3], p2)
        mm_block(qa(p1))
        s0 = send_to(0, col(z_ref, qa(p1)), rs1_ref.at[0], p1)
        mm_block(qb(p2))
        s2 = send_to(2, col(z_ref, qb(p2)), rs1_ref.at[2], p2)

        mm_block(qa(p2))
        mm_block(qb(p1))
        mm_block(qa(me))
        mm_block(qb(me))

        recv_at(1, rs1_ref.at[1]).wait_recv()
        add_block(qa(p2), rs1_ref[1])
        recv_at(3, rs1_ref.at[3]).wait_recv()
        add_block(qb(p1), rs1_ref[3])
        s4 = send_to(4, col(z_ref, qa(p2)), rs2_ref.at[0], p2)
        s5 = send_to(5, col(z_ref, qb(p1)), rs2_ref.at[1], p1)

        recv_at(0, rs1_ref.at[0]).wait_recv()
        add_block(qa(me), rs1_ref[0])
        recv_at(2, rs1_ref.at[2]).wait_recv()
        add_block(qb(me), rs1_ref[2])
        recv_at(4, rs2_ref.at[0]).wait_recv()
        add_block(qa(me), rs2_ref[0])
        recv_at(5, rs2_ref.at[1]).wait_recv()
        add_block(qb(me), rs2_ref[1])

        zb = z_ref[:, pl.ds(me * 256, 2 * C)]
        out_ref[:, pl.ds(me * 256, 2 * C)] = zb / (1.0 + jnp.exp(-zb))

        s6 = send_to(6, col(out_ref, qa(me)), col(out_ref, qa(me)), p2)
        s8 = send_to(8, col(out_ref, qa(me)), col(out_ref, qa(me)), p1)
        s7 = send_to(7, col(out_ref, qb(me)), col(out_ref, qb(me)), p1)
        s10 = send_to(10, col(out_ref, qb(me)), col(out_ref, qb(me)), p2)
        recv_at(6, col(out_ref, qa(p2))).wait_recv()
        s9 = send_to(9, col(out_ref, qa(p2)), col(out_ref, qa(p2)), p1)
        recv_at(7, col(out_ref, qb(p1))).wait_recv()
        s11 = send_to(11, col(out_ref, qb(p1)), col(out_ref, qb(p1)), p2)
        recv_at(8, col(out_ref, qa(p1))).wait_recv()
        recv_at(9, col(out_ref, qa(3 - p1))).wait_recv()
        recv_at(10, col(out_ref, qb(p2))).wait_recv()
        recv_at(11, col(out_ref, qb(p2 ^ 1))).wait_recv()

        for s in (s0, s1, s2, s3, s4, s5, s6, s7, s8, s9, s10, s11):
            s.wait_send()

    return pl.pallas_call(
        body,
        out_shape=jax.ShapeDtypeStruct((m, n), jnp.float32),
        in_specs=[
            pl.BlockSpec(memory_space=pltpu.VMEM),
            pl.BlockSpec(memory_space=pltpu.VMEM),
        ],
        out_specs=pl.BlockSpec(memory_space=pltpu.VMEM),
        scratch_shapes=[
            pltpu.VMEM((m, n), jnp.float32),
            pltpu.VMEM((4, m, C), jnp.float32),
            pltpu.VMEM((2, m, C), jnp.float32),
            pltpu.SemaphoreType.DMA((12,)),
            pltpu.SemaphoreType.DMA((12,)),
        ],
        compiler_params=pltpu.CompilerParams(collective_id=0),
    )(A, B)


# device time: 32471 ns/iter; 4.6808x vs baseline; 1.5152x over previous
import jax
import jax.numpy as jnp
from jax import lax
from jax.experimental import pallas as pl
from jax.experimental.pallas import tpu as pltpu

N_DEV = 4
C = 128
BF = jnp.bfloat16


def kernel(A, B):
    m, k = A.shape
    _, n = B.shape

    def body(a_ref, b_ref, out_ref, z_ref, sbuf, rs1_l, rs2_l,
             ag_own, ag_l, ss, rs_):
        me = lax.axis_index("i")
        p1 = me ^ 1
        p2 = 3 - me

        def qa(q):
            return q * 256

        def qb(q):
            return q * 256 + C

        def zcol(start):
            return z_ref.at[:, pl.ds(start, C)]

        def mm_block(c):
            z_ref[:, pl.ds(c, C)] = jnp.dot(
                a_ref[:, :], b_ref[:, pl.ds(c, C)],
                preferred_element_type=jnp.float32)

        def down(slot, c):
            sbuf[slot, :, :] = z_ref[:, pl.ds(c, C)].astype(BF)

        def add_block(c, src_bf):
            z_ref[:, pl.ds(c, C)] = (
                z_ref[:, pl.ds(c, C)] + src_bf.astype(jnp.float32))

        def up(c, src_bf):
            out_ref[:, pl.ds(c, C)] = src_bf.astype(jnp.float32)

        def send_to(i, src_sl, dst_sl, dev):
            d = pltpu.make_async_remote_copy(
                src_ref=src_sl, dst_ref=dst_sl,
                send_sem=ss.at[i], recv_sem=rs_.at[i],
                device_id=(dev,), device_id_type=pl.DeviceIdType.MESH,
            )
            d.start()
            return d

        def recv_at(i, dst_sl):
            return pltpu.make_async_remote_copy(
                src_ref=dst_sl, dst_ref=dst_sl,
                send_sem=ss.at[i], recv_sem=rs_.at[i],
                device_id=(me,), device_id_type=pl.DeviceIdType.MESH,
            )

        barrier_sem = pltpu.get_barrier_semaphore()
        for nbr in [p1, p2]:
            pl.semaphore_signal(
                barrier_sem, inc=1,
                device_id=(nbr,), device_id_type=pl.DeviceIdType.MESH,
            )
        pl.semaphore_wait(barrier_sem, 2)

        mm_block(qa(3 - p1))
        down(0, qa(3 - p1))
        s1 = send_to(1, sbuf.at[0], rs1_l.at[1], p1)
        mm_block(qb(p2 ^ 1))
        down(1, qb(p2 ^ 1))
        s3 = send_to(3, sbuf.at[1], rs1_l.at[3], p2)
        mm_block(qa(p1))
        down(2, qa(p1))
        s0 = send_to(0, sbuf.at[2], rs1_l.at[0], p1)
        mm_block(qb(p2))
        down(3, qb(p2))
        s2 = send_to(2, sbuf.at[3], rs1_l.at[2], p2)

        mm_block(qa(p2))
        mm_block(qb(p1))
        mm_block(qa(me))
        mm_block(qb(me))

        recv_at(1, rs1_l.at[1]).wait_recv()
        add_block(qa(p2), rs1_l[1])
        down(4, qa(p2))
        s4 = send_to(4, sbuf.at[4], rs2_l.at[0], p2)
        recv_at(3, rs1_l.at[3]).wait_recv()
        add_block(qb(p1), rs1_l[3])
        down(5, qb(p1))
        s5 = send_to(5, sbuf.at[5], rs2_l.at[1], p1)

        recv_at(0, rs1_l.at[0]).wait_recv()
        add_block(qa(me), rs1_l[0])
        recv_at(2, rs1_l.at[2]).wait_recv()
        add_block(qb(me), rs1_l[2])
        recv_at(4, rs2_l.at[0]).wait_recv()
        add_block(qa(me), rs2_l[0])
        recv_at(5, rs2_l.at[1]).wait_recv()
        add_block(qb(me), rs2_l[1])

        zb = z_ref[:, pl.ds(me * 256, 2 * C)]
        sil = zb / (1.0 + jnp.exp(-zb))
        out_ref[:, pl.ds(me * 256, 2 * C)] = sil
        ag_own[:, :] = sil.astype(BF)
        ao = ag_own.at[:, 0:C]
        bo = ag_own.at[:, C:2 * C]

        s6 = send_to(6, ao, ag_l.at[0], p2)
        s8 = send_to(8, ao, ag_l.at[2], p1)
        s7 = send_to(7, bo, ag_l.at[1], p1)
        s10 = send_to(10, bo, ag_l.at[3], p2)
        recv_at(6, ag_l.at[0]).wait_recv()
        s9 = send_to(9, ag_l.at[0], ag_l.at[4], p1)
        recv_at(7, ag_l.at[1]).wait_recv()
        s11 = send_to(11, ag_l.at[1], ag_l.at[5], p2)
        up(qa(p2), ag_l[0])
        up(qb(p1), ag_l[1])
        recv_at(8, ag_l.at[2]).wait_recv()
        up(qa(p1), ag_l[2])
        recv_at(10, ag_l.at[3]).wait_recv()
        up(qb(p2), ag_l[3])
        recv_at(9, ag_l.at[4]).wait_recv()
        up(qa(3 - p1), ag_l[4])
        recv_at(11, ag_l.at[5]).wait_recv()
        up(qb(p2 ^ 1), ag_l[5])

        for s in (s0, s1, s2, s3, s4, s5, s6, s7, s8, s9, s10, s11):
            s.wait_send()

    return pl.pallas_call(
        body,
        out_shape=jax.ShapeDtypeStruct((m, n), jnp.float32),
        in_specs=[
            pl.BlockSpec(memory_space=pltpu.VMEM),
            pl.BlockSpec(memory_space=pltpu.VMEM),
        ],
        out_specs=pl.BlockSpec(memory_space=pltpu.VMEM),
        scratch_shapes=[
            pltpu.VMEM((m, n), jnp.float32),
            pltpu.VMEM((6, m, C), BF),
            pltpu.VMEM((4, m, C), BF),
            pltpu.VMEM((2, m, C), BF),
            pltpu.VMEM((m, 2 * C), BF),
            pltpu.VMEM((6, m, C), BF),
            pltpu.SemaphoreType.DMA((12,)),
            pltpu.SemaphoreType.DMA((12,)),
        ],
        compiler_params=pltpu.CompilerParams(collective_id=0),
    )(A, B)


# device time: 30524 ns/iter; 4.9794x vs baseline; 1.0638x over previous
import jax
import jax.numpy as jnp
from jax import lax
from jax.experimental import pallas as pl
from jax.experimental.pallas import tpu as pltpu

N_DEV = 4
C = 128
BF = jnp.bfloat16


def kernel(A, B):
    m, k = A.shape
    _, n = B.shape

    def body(a_ref, b_ref, out_ref, z_ref, sbuf, rs1_l, rs2_l,
             ag_own, ag_l, ss, rs_):
        me = lax.axis_index("i")
        p1 = me ^ 1
        p2 = 3 - me

        def qa(q):
            return q * 256

        def qb(q):
            return q * 256 + C

        def zcol(start):
            return z_ref.at[:, pl.ds(start, C)]

        def down(slot, c):
            sbuf[slot, :, :] = z_ref[:, pl.ds(c, C)].astype(BF)

        def add_block(c, src_bf):
            z_ref[:, pl.ds(c, C)] = (
                z_ref[:, pl.ds(c, C)] + src_bf.astype(jnp.float32))

        def up(c, src_bf):
            out_ref[:, pl.ds(c, C)] = src_bf.astype(jnp.float32)

        def send_to(i, src_sl, dst_sl, dev):
            d = pltpu.make_async_remote_copy(
                src_ref=src_sl, dst_ref=dst_sl,
                send_sem=ss.at[i], recv_sem=rs_.at[i],
                device_id=(dev,), device_id_type=pl.DeviceIdType.MESH,
            )
            d.start()
            return d

        def recv_at(i, dst_sl):
            return pltpu.make_async_remote_copy(
                src_ref=dst_sl, dst_ref=dst_sl,
                send_sem=ss.at[i], recv_sem=rs_.at[i],
                device_id=(me,), device_id_type=pl.DeviceIdType.MESH,
            )

        barrier_sem = pltpu.get_barrier_semaphore()
        for nbr in [p1, p2]:
            pl.semaphore_signal(
                barrier_sem, inc=1,
                device_id=(nbr,), device_id_type=pl.DeviceIdType.MESH,
            )
        pl.semaphore_wait(barrier_sem, 2)

        z_ref[:, :] = jnp.dot(a_ref[:, :], b_ref[:, :],
                              preferred_element_type=jnp.float32)

        down(0, qa(3 - p1))
        s1 = send_to(1, sbuf.at[0], rs1_l.at[1], p1)
        down(1, qb(p2 ^ 1))
        s3 = send_to(3, sbuf.at[1], rs1_l.at[3], p2)
        down(2, qa(p1))
        s0 = send_to(0, sbuf.at[2], rs1_l.at[0], p1)
        down(3, qb(p2))
        s2 = send_to(2, sbuf.at[3], rs1_l.at[2], p2)

        recv_at(1, rs1_l.at[1]).wait_recv()
        add_block(qa(p2), rs1_l[1])
        down(4, qa(p2))
        s4 = send_to(4, sbuf.at[4], rs2_l.at[0], p2)
        recv_at(3, rs1_l.at[3]).wait_recv()
        add_block(qb(p1), rs1_l[3])
        down(5, qb(p1))
        s5 = send_to(5, sbuf.at[5], rs2_l.at[1], p1)

        recv_at(0, rs1_l.at[0]).wait_recv()
        add_block(qa(me), rs1_l[0])
        recv_at(2, rs1_l.at[2]).wait_recv()
        add_block(qb(me), rs1_l[2])
        recv_at(4, rs2_l.at[0]).wait_recv()
        add_block(qa(me), rs2_l[0])
        recv_at(5, rs2_l.at[1]).wait_recv()
        add_block(qb(me), rs2_l[1])

        zb = z_ref[:, pl.ds(me * 256, 2 * C)]
        sil = zb / (1.0 + jnp.exp(-zb))
        out_ref[:, pl.ds(me * 256, 2 * C)] = sil
        ag_own[:, :] = sil.astype(BF)
        ao = ag_own.at[:, 0:C]
        bo = ag_own.at[:, C:2 * C]

        s6 = send_to(6, ao, ag_l.at[0], p2)
        s7 = send_to(7, bo, ag_l.at[1], p1)
        s8 = send_to(8, ao, ag_l.at[2], p1)
        s10 = send_to(10, bo, ag_l.at[3], p2)
        recv_at(6, ag_l.at[0]).wait_recv()
        s9 = send_to(9, ag_l.at[0], ag_l.at[4], p1)
        recv_at(7, ag_l.at[1]).wait_recv()
        s11 = send_to(11, ag_l.at[1], ag_l.at[5], p2)
        up(qa(p2), ag_l[0])
        up(qb(p1), ag_l[1])
        recv_at(8, ag_l.at[2]).wait_recv()
        up(qa(p1), ag_l[2])
        recv_at(10, ag_l.at[3]).wait_recv()
        up(qb(p2), ag_l[3])
        recv_at(9, ag_l.at[4]).wait_recv()
        up(qa(3 - p1), ag_l[4])
        recv_at(11, ag_l.at[5]).wait_recv()
        up(qb(p2 ^ 1), ag_l[5])

        for s in (s0, s1, s2, s3, s4, s5, s6, s7, s8, s9, s10, s11):
            s.wait_send()

    return pl.pallas_call(
        body,
        out_shape=jax.ShapeDtypeStruct((m, n), jnp.float32),
        in_specs=[
            pl.BlockSpec(memory_space=pltpu.VMEM),
            pl.BlockSpec(memory_space=pltpu.VMEM),
        ],
        out_specs=pl.BlockSpec(memory_space=pltpu.VMEM),
        scratch_shapes=[
            pltpu.VMEM((m, n), jnp.float32),
            pltpu.VMEM((6, m, C), BF),
            pltpu.VMEM((4, m, C), BF),
            pltpu.VMEM((2, m, C), BF),
            pltpu.VMEM((m, 2 * C), BF),
            pltpu.VMEM((6, m, C), BF),
            pltpu.SemaphoreType.DMA((12,)),
            pltpu.SemaphoreType.DMA((12,)),
        ],
        compiler_params=pltpu.CompilerParams(collective_id=0),
    )(A, B)


# device time: 30375 ns/iter; 5.0038x vs baseline; 1.0049x over previous
import jax
import jax.numpy as jnp
from jax import lax
from jax.experimental import pallas as pl
from jax.experimental.pallas import tpu as pltpu

N_DEV = 4
C = 128
BF = jnp.bfloat16


def kernel(A, B):
    m, k = A.shape
    _, n = B.shape

    def body(a_ref, b_ref, out_ref, z_ref, sbuf, rs1_l, rs2_l,
             ag_own, ag_l, ss, rs_):
        me = lax.axis_index("i")
        p1 = me ^ 1
        p2 = 3 - me

        def qa(q):
            return q * 256

        def qb(q):
            return q * 256 + C

        def zcol(start):
            return z_ref.at[:, pl.ds(start, C)]

        def down(slot, c):
            sbuf[slot, :, :] = z_ref[:, pl.ds(c, C)].astype(BF)

        def add_block(c, src_bf):
            z_ref[:, pl.ds(c, C)] = (
                z_ref[:, pl.ds(c, C)] + src_bf.astype(jnp.float32))

        def up(c, src_bf):
            out_ref[:, pl.ds(c, C)] = src_bf.astype(jnp.float32)

        def send_to(i, src_sl, dst_sl, dev):
            d = pltpu.make_async_remote_copy(
                src_ref=src_sl, dst_ref=dst_sl,
                send_sem=ss.at[i], recv_sem=rs_.at[i],
                device_id=(dev,), device_id_type=pl.DeviceIdType.MESH,
            )
            d.start()
            return d

        def recv_at(i, dst_sl):
            return pltpu.make_async_remote_copy(
                src_ref=dst_sl, dst_ref=dst_sl,
                send_sem=ss.at[i], recv_sem=rs_.at[i],
                device_id=(me,), device_id_type=pl.DeviceIdType.MESH,
            )

        barrier_sem = pltpu.get_barrier_semaphore()
        for nbr in [p1, p2]:
            pl.semaphore_signal(
                barrier_sem, inc=1,
                device_id=(nbr,), device_id_type=pl.DeviceIdType.MESH,
            )
        pl.semaphore_wait(barrier_sem, 2)

        z_ref[:, :] = jnp.dot(a_ref[:, :].astype(BF), b_ref[:, :].astype(BF),
                              preferred_element_type=jnp.float32)

        down(0, qa(3 - p1))
        s1 = send_to(1, sbuf.at[0], rs1_l.at[1], p1)
        down(1, qb(p2 ^ 1))
        s3 = send_to(3, sbuf.at[1], rs1_l.at[3], p2)
        down(2, qa(p1))
        s0 = send_to(0, sbuf.at[2], rs1_l.at[0], p1)
        down(3, qb(p2))
        s2 = send_to(2, sbuf.at[3], rs1_l.at[2], p2)

        recv_at(1, rs1_l.at[1]).wait_recv()
        sbuf[4, :, :] = (z_ref[:, pl.ds(qa(p2), C)]
                         + rs1_l[1].astype(jnp.float32)).astype(BF)
        s4 = send_to(4, sbuf.at[4], rs2_l.at[0], p2)
        recv_at(3, rs1_l.at[3]).wait_recv()
        sbuf[5, :, :] = (z_ref[:, pl.ds(qb(p1), C)]
                         + rs1_l[3].astype(jnp.float32)).astype(BF)
        s5 = send_to(5, sbuf.at[5], rs2_l.at[1], p1)

        recv_at(0, rs1_l.at[0]).wait_recv()
        recv_at(2, rs1_l.at[2]).wait_recv()
        recv_at(4, rs2_l.at[0]).wait_recv()
        za = (z_ref[:, pl.ds(qa(me), C)]
              + rs1_l[0].astype(jnp.float32) + rs2_l[0].astype(jnp.float32))
        sa = za / (1.0 + jnp.exp(-za))
        out_ref[:, pl.ds(qa(me), C)] = sa
        ag_own[:, 0:C] = sa.astype(BF)
        recv_at(5, rs2_l.at[1]).wait_recv()
        zb2 = (z_ref[:, pl.ds(qb(me), C)]
               + rs1_l[2].astype(jnp.float32) + rs2_l[1].astype(jnp.float32))
        sb = zb2 / (1.0 + jnp.exp(-zb2))
        out_ref[:, pl.ds(qb(me), C)] = sb
        ag_own[:, C:2 * C] = sb.astype(BF)
        ao = ag_own.at[:, 0:C]
        bo = ag_own.at[:, C:2 * C]

        s6 = send_to(6, ao, ag_l.at[0], p2)
        s7 = send_to(7, bo, ag_l.at[1], p1)
        s8 = send_to(8, ao, ag_l.at[2], p1)
        s10 = send_to(10, bo, ag_l.at[3], p2)
        recv_at(6, ag_l.at[0]).wait_recv()
        s9 = send_to(9, ag_l.at[0], ag_l.at[4], p1)
        recv_at(7, ag_l.at[1]).wait_recv()
        s11 = send_to(11, ag_l.at[1], ag_l.at[5], p2)
        up(qa(p2), ag_l[0])
        up(qb(p1), ag_l[1])
        recv_at(8, ag_l.at[2]).wait_recv()
        up(qa(p1), ag_l[2])
        recv_at(10, ag_l.at[3]).wait_recv()
        up(qb(p2), ag_l[3])
        recv_at(9, ag_l.at[4]).wait_recv()
        up(qa(3 - p1), ag_l[4])
        recv_at(11, ag_l.at[5]).wait_recv()
        up(qb(p2 ^ 1), ag_l[5])

        for s in (s0, s1, s2, s3, s4, s5, s6, s7, s8, s9, s10, s11):
            s.wait_send()

    return pl.pallas_call(
        body,
        out_shape=jax.ShapeDtypeStruct((m, n), jnp.float32),
        in_specs=[
            pl.BlockSpec(memory_space=pltpu.VMEM),
            pl.BlockSpec(memory_space=pltpu.VMEM),
        ],
        out_specs=pl.BlockSpec(memory_space=pltpu.VMEM),
        scratch_shapes=[
            pltpu.VMEM((m, n), jnp.float32),
            pltpu.VMEM((6, m, C), BF),
            pltpu.VMEM((4, m, C), BF),
            pltpu.VMEM((2, m, C), BF),
            pltpu.VMEM((m, 2 * C), BF),
            pltpu.VMEM((6, m, C), BF),
            pltpu.SemaphoreType.DMA((12,)),
            pltpu.SemaphoreType.DMA((12,)),
        ],
        compiler_params=pltpu.CompilerParams(collective_id=0),
    )(A, B)


# device time: 30229 ns/iter; 5.0280x vs baseline; 1.0048x over previous
import jax
import jax.numpy as jnp
from jax import lax
from jax.experimental import pallas as pl
from jax.experimental.pallas import tpu as pltpu

N_DEV = 4
C = 128
H = 512


def kernel(A, B):
    m, k = A.shape
    _, n = B.shape

    def body(a_ref, b_ref, out_ref, z_ref, bbf, sbuf, rs1_l, rs2_l,
             ag_own, ag_l, ss, rs_):
        me = lax.axis_index("i")
        p1 = me ^ 1
        p2 = 3 - me
        f32 = jnp.float32
        BF = jnp.bfloat16

        def qa(q):
            return q * 256

        def qb(q):
            return q * 256 + C

        def send_to(e, h, src_sl, dst_sl, dev):
            d = pltpu.make_async_remote_copy(
                src_ref=src_sl, dst_ref=dst_sl,
                send_sem=ss.at[2 * e + h], recv_sem=rs_.at[2 * e + h],
                device_id=(dev,), device_id_type=pl.DeviceIdType.MESH,
            )
            d.start()
            return d

        def wait_recv(e, h, dst_sl):
            pltpu.make_async_remote_copy(
                src_ref=dst_sl, dst_ref=dst_sl,
                send_sem=ss.at[2 * e + h], recv_sem=rs_.at[2 * e + h],
                device_id=(me,), device_id_type=pl.DeviceIdType.MESH,
            ).wait_recv()

        barrier_sem = pltpu.get_barrier_semaphore()
        for nbr in [p1, p2]:
            pl.semaphore_signal(
                barrier_sem, inc=1,
                device_id=(nbr,), device_id_type=pl.DeviceIdType.MESH,
            )
        pl.semaphore_wait(barrier_sem, 2)

        bbf[:, :] = b_ref[:, :].astype(BF)

        sends = []
        for h in (0, 1):
            r0 = h * H
            z_ref[pl.ds(r0, H), :] = jnp.dot(
                a_ref[pl.ds(r0, H), :].astype(BF), bbf[:, :],
                preferred_element_type=f32)
            for e, slot, c in ((1, 1, qa(3 - p1)), (3, 3, qb(p2 ^ 1)),
                               (0, 0, qa(p1)), (2, 2, qb(p2))):
                sbuf[slot, pl.ds(r0, H), :] = (
                    z_ref[pl.ds(r0, H), pl.ds(c, C)].astype(BF))
                sends.append(send_to(
                    e, h,
                    sbuf.at[slot, pl.ds(r0, H), :],
                    rs1_l.at[slot, pl.ds(r0, H), :],
                    p1 if e in (0, 1) else p2))

        for h in (0, 1):
            r0 = h * H
            wait_recv(1, h, rs1_l.at[1, pl.ds(r0, H), :])
            sbuf[4, pl.ds(r0, H), :] = (
                z_ref[pl.ds(r0, H), pl.ds(qa(p2), C)]
                + rs1_l[1, pl.ds(r0, H), :].astype(f32)).astype(BF)
            sends.append(send_to(4, h, sbuf.at[4, pl.ds(r0, H), :],
                                 rs2_l.at[0, pl.ds(r0, H), :], p2))
            wait_recv(3, h, rs1_l.at[3, pl.ds(r0, H), :])
            sbuf[5, pl.ds(r0, H), :] = (
                z_ref[pl.ds(r0, H), pl.ds(qb(p1), C)]
                + rs1_l[3, pl.ds(r0, H), :].astype(f32)).astype(BF)
            sends.append(send_to(5, h, sbuf.at[5, pl.ds(r0, H), :],
                                 rs2_l.at[1, pl.ds(r0, H), :], p1))

        for h in (0, 1):
            r0 = h * H
            wait_recv(0, h, rs1_l.at[0, pl.ds(r0, H), :])
            wait_recv(4, h, rs2_l.at[0, pl.ds(r0, H), :])
            za = (z_ref[pl.ds(r0, H), pl.ds(qa(me), C)]
                  + rs1_l[0, pl.ds(r0, H), :].astype(f32)
                  + rs2_l[0, pl.ds(r0, H), :].astype(f32))
            sa = za / (1.0 + jnp.exp(-za))
            out_ref[pl.ds(r0, H), pl.ds(qa(me), C)] = sa
            ag_own[pl.ds(r0, H), 0:C] = sa.astype(BF)
            ao = ag_own.at[pl.ds(r0, H), pl.ds(0, C)]
            sends.append(send_to(6, h, ao, ag_l.at[0, pl.ds(r0, H), :], p2))
            sends.append(send_to(8, h, ao, ag_l.at[2, pl.ds(r0, H), :], p1))

            wait_recv(2, h, rs1_l.at[2, pl.ds(r0, H), :])
            wait_recv(5, h, rs2_l.at[1, pl.ds(r0, H), :])
            zb = (z_ref[pl.ds(r0, H), pl.ds(qb(me), C)]
                  + rs1_l[2, pl.ds(r0, H), :].astype(f32)
                  + rs2_l[1, pl.ds(r0, H), :].astype(f32))
            sb = zb / (1.0 + jnp.exp(-zb))
            out_ref[pl.ds(r0, H), pl.ds(qb(me), C)] = sb
            ag_own[pl.ds(r0, H), C:2 * C] = sb.astype(BF)
            bo = ag_own.at[pl.ds(r0, H), pl.ds(C, C)]
            sends.append(send_to(7, h, bo, ag_l.at[1, pl.ds(r0, H), :], p1))
            sends.append(send_to(10, h, bo, ag_l.at[3, pl.ds(r0, H), :], p2))

        for h in (0, 1):
            r0 = h * H
            wait_recv(6, h, ag_l.at[0, pl.ds(r0, H), :])
            sends.append(send_to(9, h, ag_l.at[0, pl.ds(r0, H), :],
                                 ag_l.at[4, pl.ds(r0, H), :], p1))
            wait_recv(7, h, ag_l.at[1, pl.ds(r0, H), :])
            sends.append(send_to(11, h, ag_l.at[1, pl.ds(r0, H), :],
                                 ag_l.at[5, pl.ds(r0, H), :], p2))
            out_ref[pl.ds(r0, H), pl.ds(qa(p2), C)] = (
                ag_l[0, pl.ds(r0, H), :].astype(f32))
            out_ref[pl.ds(r0, H), pl.ds(qb(p1), C)] = (
                ag_l[1, pl.ds(r0, H), :].astype(f32))
            wait_recv(8, h, ag_l.at[2, pl.ds(r0, H), :])
            out_ref[pl.ds(r0, H), pl.ds(qa(p1), C)] = (
                ag_l[2, pl.ds(r0, H), :].astype(f32))
            wait_recv(10, h, ag_l.at[3, pl.ds(r0, H), :])
            out_ref[pl.ds(r0, H), pl.ds(qb(p2), C)] = (
                ag_l[3, pl.ds(r0, H), :].astype(f32))
            wait_recv(9, h, ag_l.at[4, pl.ds(r0, H), :])
            out_ref[pl.ds(r0, H), pl.ds(qa(3 - p1), C)] = (
                ag_l[4, pl.ds(r0, H), :].astype(f32))
            wait_recv(11, h, ag_l.at[5, pl.ds(r0, H), :])
            out_ref[pl.ds(r0, H), pl.ds(qb(p2 ^ 1), C)] = (
                ag_l[5, pl.ds(r0, H), :].astype(f32))

        for s in sends:
            s.wait_send()

    return pl.pallas_call(
        body,
        out_shape=jax.ShapeDtypeStruct((m, n), jnp.float32),
        in_specs=[
            pl.BlockSpec(memory_space=pltpu.VMEM),
            pl.BlockSpec(memory_space=pltpu.VMEM),
        ],
        out_specs=pl.BlockSpec(memory_space=pltpu.VMEM),
        scratch_shapes=[
            pltpu.VMEM((m, n), jnp.float32),
            pltpu.VMEM((k, n), jnp.bfloat16),
            pltpu.VMEM((6, m, C), jnp.bfloat16),
            pltpu.VMEM((4, m, C), jnp.bfloat16),
            pltpu.VMEM((2, m, C), jnp.bfloat16),
            pltpu.VMEM((m, 2 * C), jnp.bfloat16),
            pltpu.VMEM((6, m, C), jnp.bfloat16),
            pltpu.SemaphoreType.DMA((24,)),
            pltpu.SemaphoreType.DMA((24,)),
        ],
        compiler_params=pltpu.CompilerParams(collective_id=0),
    )(A, B)


# device time: 30210 ns/iter; 5.0311x vs baseline; 1.0006x over previous
import jax
import jax.numpy as jnp
from jax import lax
from jax.experimental import pallas as pl
from jax.experimental.pallas import tpu as pltpu

N_DEV = 4
C = 128
H = 512


def kernel(A, B):
    m, k = A.shape
    _, n = B.shape

    def body(a_ref, b_ref, out_ref, z_ref, bbf, sbuf, rs1_l, rs2_l,
             ag_own, ag_l, ss, rs_):
        me = lax.axis_index("i")
        p1 = me ^ 1
        p2 = 3 - me
        f32 = jnp.float32
        BF = jnp.bfloat16

        def qa(q):
            return q * 256

        def qb(q):
            return q * 256 + C

        def send_to(e, h, src_sl, dst_sl, dev):
            d = pltpu.make_async_remote_copy(
                src_ref=src_sl, dst_ref=dst_sl,
                send_sem=ss.at[2 * e + h], recv_sem=rs_.at[2 * e + h],
                device_id=(dev,), device_id_type=pl.DeviceIdType.MESH,
            )
            d.start()
            return d

        def wait_recv(e, h, dst_sl):
            pltpu.make_async_remote_copy(
                src_ref=dst_sl, dst_ref=dst_sl,
                send_sem=ss.at[2 * e + h], recv_sem=rs_.at[2 * e + h],
                device_id=(me,), device_id_type=pl.DeviceIdType.MESH,
            ).wait_recv()

        barrier_sem = pltpu.get_barrier_semaphore()
        for nbr in [p1, p2]:
            pl.semaphore_signal(
                barrier_sem, inc=1,
                device_id=(nbr,), device_id_type=pl.DeviceIdType.MESH,
            )
        pl.semaphore_wait(barrier_sem, 2)

        bbf[:, :] = b_ref[:, :].astype(BF)

        sends = []
        for h in (0, 1):
            r0 = h * H
            z_ref[pl.ds(r0, H), :] = jnp.dot(
                a_ref[pl.ds(r0, H), :].astype(BF), bbf[:, :],
                preferred_element_type=f32).astype(BF)
            for e, slot, c in ((1, 1, qa(3 - p1)), (3, 3, qb(p2 ^ 1)),
                               (0, 0, qa(p1)), (2, 2, qb(p2))):
                sends.append(send_to(
                    e, h,
                    z_ref.at[pl.ds(r0, H), pl.ds(c, C)],
                    rs1_l.at[slot, pl.ds(r0, H), :],
                    p1 if e in (0, 1) else p2))

        for h in (0, 1):
            r0 = h * H
            wait_recv(1, h, rs1_l.at[1, pl.ds(r0, H), :])
            sbuf[0, pl.ds(r0, H), :] = (
                z_ref[pl.ds(r0, H), pl.ds(qa(p2), C)]
                + rs1_l[1, pl.ds(r0, H), :])
            sends.append(send_to(4, h, sbuf.at[0, pl.ds(r0, H), :],
                                 rs2_l.at[0, pl.ds(r0, H), :], p2))
            wait_recv(3, h, rs1_l.at[3, pl.ds(r0, H), :])
            sbuf[1, pl.ds(r0, H), :] = (
                z_ref[pl.ds(r0, H), pl.ds(qb(p1), C)]
                + rs1_l[3, pl.ds(r0, H), :])
            sends.append(send_to(5, h, sbuf.at[1, pl.ds(r0, H), :],
                                 rs2_l.at[1, pl.ds(r0, H), :], p1))

        for h in (0, 1):
            r0 = h * H
            wait_recv(0, h, rs1_l.at[0, pl.ds(r0, H), :])
            wait_recv(4, h, rs2_l.at[0, pl.ds(r0, H), :])
            za = (z_ref[pl.ds(r0, H), pl.ds(qa(me), C)].astype(f32)
                  + rs1_l[0, pl.ds(r0, H), :].astype(f32)
                  + rs2_l[0, pl.ds(r0, H), :].astype(f32))
            sa = za / (1.0 + jnp.exp(-za))
            out_ref[pl.ds(r0, H), pl.ds(qa(me), C)] = sa
            ag_own[pl.ds(r0, H), 0:C] = sa.astype(BF)
            ao = ag_own.at[pl.ds(r0, H), pl.ds(0, C)]
            sends.append(send_to(6, h, ao, ag_l.at[0, pl.ds(r0, H), :], p2))
            sends.append(send_to(8, h, ao, ag_l.at[2, pl.ds(r0, H), :], p1))

            wait_recv(2, h, rs1_l.at[2, pl.ds(r0, H), :])
            wait_recv(5, h, rs2_l.at[1, pl.ds(r0, H), :])
            zb = (z_ref[pl.ds(r0, H), pl.ds(qb(me), C)].astype(f32)
                  + rs1_l[2, pl.ds(r0, H), :].astype(f32)
                  + rs2_l[1, pl.ds(r0, H), :].astype(f32))
            sb = zb / (1.0 + jnp.exp(-zb))
            out_ref[pl.ds(r0, H), pl.ds(qb(me), C)] = sb
            ag_own[pl.ds(r0, H), C:2 * C] = sb.astype(BF)
            bo = ag_own.at[pl.ds(r0, H), pl.ds(C, C)]
            sends.append(send_to(7, h, bo, ag_l.at[1, pl.ds(r0, H), :], p1))
            sends.append(send_to(10, h, bo, ag_l.at[3, pl.ds(r0, H), :], p2))

        for h in (0, 1):
            r0 = h * H
            wait_recv(6, h, ag_l.at[0, pl.ds(r0, H), :])
            sends.append(send_to(9, h, ag_l.at[0, pl.ds(r0, H), :],
                                 ag_l.at[4, pl.ds(r0, H), :], p1))
            wait_recv(7, h, ag_l.at[1, pl.ds(r0, H), :])
            sends.append(send_to(11, h, ag_l.at[1, pl.ds(r0, H), :],
                                 ag_l.at[5, pl.ds(r0, H), :], p2))
            out_ref[pl.ds(r0, H), pl.ds(qa(p2), C)] = (
                ag_l[0, pl.ds(r0, H), :].astype(f32))
            out_ref[pl.ds(r0, H), pl.ds(qb(p1), C)] = (
                ag_l[1, pl.ds(r0, H), :].astype(f32))
            wait_recv(8, h, ag_l.at[2, pl.ds(r0, H), :])
            out_ref[pl.ds(r0, H), pl.ds(qa(p1), C)] = (
                ag_l[2, pl.ds(r0, H), :].astype(f32))
            wait_recv(10, h, ag_l.at[3, pl.ds(r0, H), :])
            out_ref[pl.ds(r0, H), pl.ds(qb(p2), C)] = (
                ag_l[3, pl.ds(r0, H), :].astype(f32))
            wait_recv(9, h, ag_l.at[4, pl.ds(r0, H), :])
            out_ref[pl.ds(r0, H), pl.ds(qa(3 - p1), C)] = (
                ag_l[4, pl.ds(r0, H), :].astype(f32))
            wait_recv(11, h, ag_l.at[5, pl.ds(r0, H), :])
            out_ref[pl.ds(r0, H), pl.ds(qb(p2 ^ 1), C)] = (
                ag_l[5, pl.ds(r0, H), :].astype(f32))

        for s in sends:
            s.wait_send()

    return pl.pallas_call(
        body,
        out_shape=jax.ShapeDtypeStruct((m, n), jnp.float32),
        in_specs=[
            pl.BlockSpec(memory_space=pltpu.VMEM),
            pl.BlockSpec(memory_space=pltpu.VMEM),
        ],
        out_specs=pl.BlockSpec(memory_space=pltpu.VMEM),
        scratch_shapes=[
            pltpu.VMEM((m, n), jnp.bfloat16),
            pltpu.VMEM((k, n), jnp.bfloat16),
            pltpu.VMEM((2, m, C), jnp.bfloat16),
            pltpu.VMEM((4, m, C), jnp.bfloat16),
            pltpu.VMEM((2, m, C), jnp.bfloat16),
            pltpu.VMEM((m, 2 * C), jnp.bfloat16),
            pltpu.VMEM((6, m, C), jnp.bfloat16),
            pltpu.SemaphoreType.DMA((24,)),
            pltpu.SemaphoreType.DMA((24,)),
        ],
        compiler_params=pltpu.CompilerParams(collective_id=0),
    )(A, B)
